# baseline (speedup 1.0000x reference)
import functools

import jax
import jax.numpy as jnp
from jax import lax
from jax.experimental import pallas as pl
from jax.experimental.pallas import tpu as pltpu
from jax.experimental.pallas import tpu_sc as plsc

B = 8
N = 2048
L = 16
NKV = N // L
QCHUNK = N // 2
QBLK = 4
UNROLL = 8


def _round_bf16(v):
    u = plsc.bitcast(v, jnp.uint32)
    lsb = (u >> jnp.uint32(16)) & jnp.uint32(1)
    r = (u + jnp.uint32(0x7FFF) + lsb) & jnp.uint32(0xFFFF0000)
    return plsc.bitcast(r, jnp.float32)


def _tile_body(pts_hbm, out_hbm, qx, qy, qz, q2, kx, ky, kz, k2, res):
    c = lax.axis_index("c")
    s = lax.axis_index("s")
    w = s * 2 + c
    d = w // 16
    r = w % 16
    b = r // 2
    h = r % 2
    q0 = h * QCHUNK
    dk = 1 - d

    pltpu.sync_copy(pts_hbm.at[d, 0, b, pl.ds(q0, QCHUNK)], qx)
    pltpu.sync_copy(pts_hbm.at[d, 1, b, pl.ds(q0, QCHUNK)], qy)
    pltpu.sync_copy(pts_hbm.at[d, 2, b, pl.ds(q0, QCHUNK)], qz)
    pltpu.sync_copy(pts_hbm.at[dk, 0, b], kx)
    pltpu.sync_copy(pts_hbm.at[dk, 1, b], ky)
    pltpu.sync_copy(pts_hbm.at[dk, 2, b], kz)

    def k2_body(j, carry):
        off = j * L
        xs = kx[pl.ds(off, L)]
        ys = ky[pl.ds(off, L)]
        zs = kz[pl.ds(off, L)]
        k2[pl.ds(off, L)] = xs * xs + ys * ys + zs * zs
        kx[pl.ds(off, L)] = _round_bf16(xs)
        ky[pl.ds(off, L)] = _round_bf16(ys)
        kz[pl.ds(off, L)] = _round_bf16(zs)
        return carry

    lax.fori_loop(0, NKV, k2_body, 0)

    def q2_body(j, carry):
        off = j * L
        xs = qx[pl.ds(off, L)]
        ys = qy[pl.ds(off, L)]
        zs = qz[pl.ds(off, L)]
        q2[pl.ds(off, L)] = xs * xs + ys * ys + zs * zs
        qx[pl.ds(off, L)] = _round_bf16(xs)
        qy[pl.ds(off, L)] = _round_bf16(ys)
        qz[pl.ds(off, L)] = _round_bf16(zs)
        return carry

    lax.fori_loop(0, QCHUNK // L, q2_body, 0)

    inf = jnp.full((L,), jnp.inf, jnp.float32)

    def qblk_body(i, carry):
        qbase = i * QBLK
        av = []
        bv = []
        cv = []
        q2v = []
        for u in range(QBLK):
            idx = jnp.full((L,), qbase + u, jnp.int32)
            av.append(-2.0 * plsc.load_gather(qx, [idx]))
            bv.append(-2.0 * plsc.load_gather(qy, [idx]))
            cv.append(-2.0 * plsc.load_gather(qz, [idx]))
            q2v.append(plsc.load_gather(q2, [idx]))

        def k_body(jj, accs):
            accs = list(accs)
            for t in range(UNROLL):
                off = (jj * UNROLL + t) * L
                xk = kx[pl.ds(off, L)]
                yk = ky[pl.ds(off, L)]
                zk = kz[pl.ds(off, L)]
                kk = k2[pl.ds(off, L)]
                for u in range(QBLK):
                    dv = kk + av[u] * xk + bv[u] * yk + cv[u] * zk
                    accs[u] = jnp.minimum(accs[u], dv)
            return tuple(accs)

        accs = lax.fori_loop(0, NKV // UNROLL, k_body,
                             (inf,) * QBLK)
        lane0 = lax.iota(jnp.int32, L) == 0
        for u in range(QBLK):
            m = jnp.min(accs[u] + q2v[u])
            idx = jnp.full((L,), qbase + u, jnp.int32)
            plsc.store_scatter(res, [idx], jnp.full((L,), m), mask=lane0)
        return carry

    lax.fori_loop(0, QCHUNK // QBLK, qblk_body, 0)

    pltpu.sync_copy(res, out_hbm.at[d, b, pl.ds(q0, QCHUNK)])


_mesh = plsc.VectorSubcoreMesh(core_axis_name="c", subcore_axis_name="s")

_sc_chamfer = pl.kernel(
    _tile_body,
    out_type=jax.ShapeDtypeStruct((2, B, N), jnp.float32),
    mesh=_mesh,
    scratch_types=[
        pltpu.VMEM((QCHUNK,), jnp.float32),
        pltpu.VMEM((QCHUNK,), jnp.float32),
        pltpu.VMEM((QCHUNK,), jnp.float32),
        pltpu.VMEM((QCHUNK,), jnp.float32),
        pltpu.VMEM((N,), jnp.float32),
        pltpu.VMEM((N,), jnp.float32),
        pltpu.VMEM((N,), jnp.float32),
        pltpu.VMEM((N,), jnp.float32),
        pltpu.VMEM((QCHUNK,), jnp.float32),
    ],
    compiler_params=pltpu.CompilerParams(needs_layout_passes=False),
)


def kernel(xyz1, xyz2):
    pts = jnp.stack(
        [jnp.moveaxis(xyz1, -1, 0), jnp.moveaxis(xyz2, -1, 0)]
    )
    out = _sc_chamfer(pts)
    return out[0], out[1]


# speedup vs baseline: 1.8335x; 1.8335x over previous
import functools

import jax
import jax.numpy as jnp
from jax import lax
from jax.experimental import pallas as pl
from jax.experimental.pallas import tpu as pltpu
from jax.experimental.pallas import tpu_sc as plsc

B = 8
N = 2048
L = 16
NKV = N // L
QCHUNK = N // 2
QBLK = 4
UNROLL = 4


def _round_bf16(v):
    u = plsc.bitcast(v, jnp.uint32)
    lsb = (u >> jnp.uint32(16)) & jnp.uint32(1)
    r = (u + jnp.uint32(0x7FFF) + lsb) & jnp.uint32(0xFFFF0000)
    return plsc.bitcast(r, jnp.float32)


def _tile_body(pts_hbm, out_hbm, qx, qy, qz, q2, kx, ky, kz, k2, res):
    c = lax.axis_index("c")
    s = lax.axis_index("s")
    w = s * 2 + c
    d = w // 16
    r = w % 16
    b = r // 2
    h = r % 2
    q0 = h * QCHUNK
    dk = 1 - d

    pltpu.sync_copy(pts_hbm.at[d, 0, b, pl.ds(q0, QCHUNK)], qx)
    pltpu.sync_copy(pts_hbm.at[d, 1, b, pl.ds(q0, QCHUNK)], qy)
    pltpu.sync_copy(pts_hbm.at[d, 2, b, pl.ds(q0, QCHUNK)], qz)
    pltpu.sync_copy(pts_hbm.at[dk, 0, b], kx)
    pltpu.sync_copy(pts_hbm.at[dk, 1, b], ky)
    pltpu.sync_copy(pts_hbm.at[dk, 2, b], kz)

    def k2_body(j, carry):
        off = j * L
        xs = kx[pl.ds(off, L)]
        ys = ky[pl.ds(off, L)]
        zs = kz[pl.ds(off, L)]
        k2[pl.ds(off, L)] = xs * xs + ys * ys + zs * zs
        kx[pl.ds(off, L)] = _round_bf16(xs)
        ky[pl.ds(off, L)] = _round_bf16(ys)
        kz[pl.ds(off, L)] = _round_bf16(zs)
        return carry

    lax.fori_loop(0, NKV, k2_body, 0)

    def q2_body(j, carry):
        off = j * L
        xs = qx[pl.ds(off, L)]
        ys = qy[pl.ds(off, L)]
        zs = qz[pl.ds(off, L)]
        q2[pl.ds(off, L)] = xs * xs + ys * ys + zs * zs
        qx[pl.ds(off, L)] = _round_bf16(xs)
        qy[pl.ds(off, L)] = _round_bf16(ys)
        qz[pl.ds(off, L)] = _round_bf16(zs)
        return carry

    lax.fori_loop(0, QCHUNK // L, q2_body, 0)

    inf = jnp.full((L,), jnp.inf, jnp.float32)

    def qblk_body(i, carry):
        qbase = i * QBLK
        av = []
        bv = []
        cv = []
        q2v = []
        for u in range(QBLK):
            idx = jnp.full((L,), qbase + u, jnp.int32)
            av.append(-2.0 * plsc.load_gather(qx, [idx]))
            bv.append(-2.0 * plsc.load_gather(qy, [idx]))
            cv.append(-2.0 * plsc.load_gather(qz, [idx]))
            q2v.append(plsc.load_gather(q2, [idx]))

        def k_body(jj, accs):
            accs = list(accs)
            for t in range(UNROLL):
                off = (jj * UNROLL + t) * L
                xk = kx[pl.ds(off, L)]
                yk = ky[pl.ds(off, L)]
                zk = kz[pl.ds(off, L)]
                kk = k2[pl.ds(off, L)]
                for u in range(QBLK):
                    dv = kk + av[u] * xk + bv[u] * yk + cv[u] * zk
                    accs[u] = jnp.minimum(accs[u], dv)
            return tuple(accs)

        accs = lax.fori_loop(0, NKV // UNROLL, k_body,
                             (inf,) * QBLK)
        lane0 = lax.iota(jnp.int32, L) == 0
        for u in range(QBLK):
            m = jnp.min(accs[u] + q2v[u])
            idx = jnp.full((L,), qbase + u, jnp.int32)
            plsc.store_scatter(res, [idx], jnp.full((L,), m), mask=lane0)
        return carry

    lax.fori_loop(0, QCHUNK // QBLK, qblk_body, 0)

    pltpu.sync_copy(res, out_hbm.at[d, b, pl.ds(q0, QCHUNK)])


_mesh = plsc.VectorSubcoreMesh(core_axis_name="c", subcore_axis_name="s")

_sc_chamfer = pl.kernel(
    _tile_body,
    out_type=jax.ShapeDtypeStruct((2, B, N), jnp.float32),
    mesh=_mesh,
    scratch_types=[
        pltpu.VMEM((QCHUNK,), jnp.float32),
        pltpu.VMEM((QCHUNK,), jnp.float32),
        pltpu.VMEM((QCHUNK,), jnp.float32),
        pltpu.VMEM((QCHUNK,), jnp.float32),
        pltpu.VMEM((N,), jnp.float32),
        pltpu.VMEM((N,), jnp.float32),
        pltpu.VMEM((N,), jnp.float32),
        pltpu.VMEM((N,), jnp.float32),
        pltpu.VMEM((QCHUNK,), jnp.float32),
    ],
    compiler_params=pltpu.CompilerParams(needs_layout_passes=False),
)


def kernel(xyz1, xyz2):
    pts = jnp.stack(
        [jnp.moveaxis(xyz1, -1, 0), jnp.moveaxis(xyz2, -1, 0)]
    )
    out = _sc_chamfer(pts)
    return out[0], out[1]


# speedup vs baseline: 5.1619x; 2.8153x over previous
import functools

import jax
import jax.numpy as jnp
from jax import lax
from jax.experimental import pallas as pl
from jax.experimental.pallas import tpu as pltpu
from jax.experimental.pallas import tpu_sc as plsc

B = 8
N = 2048
L = 16
NKV = N // L
QBLK = 4
UNROLL = 4

KSC = 1

TILE_N = 256
NB = N // TILE_N


_CHUNKS = 16 // KSC if KSC else 1
_QLEN = N // _CHUNKS if KSC else N


def _round_bf16(v):
    u = plsc.bitcast(v, jnp.uint32)
    lsb = (u >> jnp.uint32(16)) & jnp.uint32(1)
    r = (u + jnp.uint32(0x7FFF) + lsb) & jnp.uint32(0xFFFF0000)
    return plsc.bitcast(r, jnp.float32)


def _sc_tile_body(pts_hbm, out_hbm, qx, qy, qz, q2, kx, ky, kz, k2, res):
    c = lax.axis_index("c")
    s = lax.axis_index("s")
    w = s * 2 + c
    d = w // 16
    r = w % 16
    b = r // _CHUNKS
    h = r % _CHUNKS
    q0 = h * _QLEN
    dk = 1 - d

    pltpu.sync_copy(pts_hbm.at[d, 0, b, pl.ds(q0, _QLEN)], qx)
    pltpu.sync_copy(pts_hbm.at[d, 1, b, pl.ds(q0, _QLEN)], qy)
    pltpu.sync_copy(pts_hbm.at[d, 2, b, pl.ds(q0, _QLEN)], qz)
    pltpu.sync_copy(pts_hbm.at[dk, 0, b], kx)
    pltpu.sync_copy(pts_hbm.at[dk, 1, b], ky)
    pltpu.sync_copy(pts_hbm.at[dk, 2, b], kz)

    def k2_body(j, carry):
        off = j * L
        xs = kx[pl.ds(off, L)]
        ys = ky[pl.ds(off, L)]
        zs = kz[pl.ds(off, L)]
        k2[pl.ds(off, L)] = xs * xs + ys * ys + zs * zs
        kx[pl.ds(off, L)] = _round_bf16(xs)
        ky[pl.ds(off, L)] = _round_bf16(ys)
        kz[pl.ds(off, L)] = _round_bf16(zs)
        return carry

    lax.fori_loop(0, NKV, k2_body, 0)

    def q2_body(j, carry):
        off = j * L
        xs = qx[pl.ds(off, L)]
        ys = qy[pl.ds(off, L)]
        zs = qz[pl.ds(off, L)]
        q2[pl.ds(off, L)] = xs * xs + ys * ys + zs * zs
        qx[pl.ds(off, L)] = _round_bf16(xs)
        qy[pl.ds(off, L)] = _round_bf16(ys)
        qz[pl.ds(off, L)] = _round_bf16(zs)
        return carry

    lax.fori_loop(0, _QLEN // L, q2_body, 0)

    inf = jnp.full((L,), jnp.inf, jnp.float32)

    def qblk_body(i, carry):
        qbase = i * QBLK
        av = []
        bv = []
        cv = []
        q2v = []
        for u in range(QBLK):
            idx = jnp.full((L,), qbase + u, jnp.int32)
            av.append(-2.0 * plsc.load_gather(qx, [idx]))
            bv.append(-2.0 * plsc.load_gather(qy, [idx]))
            cv.append(-2.0 * plsc.load_gather(qz, [idx]))
            q2v.append(plsc.load_gather(q2, [idx]))

        def k_body(jj, accs):
            accs = list(accs)
            for t in range(UNROLL):
                off = (jj * UNROLL + t) * L
                xk = kx[pl.ds(off, L)]
                yk = ky[pl.ds(off, L)]
                zk = kz[pl.ds(off, L)]
                kk = k2[pl.ds(off, L)]
                for u in range(QBLK):
                    dv = kk + av[u] * xk + bv[u] * yk + cv[u] * zk
                    accs[u] = jnp.minimum(accs[u], dv)
            return tuple(accs)

        accs = lax.fori_loop(0, NKV // UNROLL, k_body,
                             (inf,) * QBLK)
        lane0 = lax.iota(jnp.int32, L) == 0
        for u in range(QBLK):
            m = jnp.min(accs[u] + q2v[u])
            idx = jnp.full((L,), qbase + u, jnp.int32)
            plsc.store_scatter(res, [idx], jnp.full((L,), m), mask=lane0)
        return carry

    lax.fori_loop(0, _QLEN // QBLK, qblk_body, 0)

    pltpu.sync_copy(res, out_hbm.at[d, b, pl.ds(q0, _QLEN)])


def _make_sc_kernel():
    mesh = plsc.VectorSubcoreMesh(core_axis_name="c", subcore_axis_name="s")
    return pl.kernel(
        _sc_tile_body,
        out_type=jax.ShapeDtypeStruct((2, KSC, N), jnp.float32),
        mesh=mesh,
        scratch_types=[
            pltpu.VMEM((_QLEN,), jnp.float32),
            pltpu.VMEM((_QLEN,), jnp.float32),
            pltpu.VMEM((_QLEN,), jnp.float32),
            pltpu.VMEM((_QLEN,), jnp.float32),
            pltpu.VMEM((N,), jnp.float32),
            pltpu.VMEM((N,), jnp.float32),
            pltpu.VMEM((N,), jnp.float32),
            pltpu.VMEM((N,), jnp.float32),
            pltpu.VMEM((_QLEN,), jnp.float32),
        ],
        compiler_params=pltpu.CompilerParams(needs_layout_passes=False),
    )


def _tc_body(x1_ref, p2t_ref, d1_ref, d2_ref):
    i = pl.program_id(1)
    xb = x1_ref[0]
    yt = p2t_ref[0]
    n1 = jnp.sum(xb * xb, axis=1, keepdims=True)
    n2 = jnp.sum(yt * yt, axis=0, keepdims=True)
    a = (-2.0 * xb).astype(jnp.bfloat16)
    yr = yt.astype(jnp.bfloat16)
    t = lax.dot_general(a, yr, (((1,), (0,)), ((), ())),
                        preferred_element_type=jnp.float32)
    d1_ref[0, 0, 0, :] = jnp.min(t + n2, axis=1) + n1[:, 0]
    v = jnp.min(t + n1, axis=0, keepdims=True)

    @pl.when(i == 0)
    def _():
        d2_ref[0] = v

    @pl.when(i > 0)
    def _():
        d2_ref[0] = jnp.minimum(d2_ref[0], v)

    @pl.when(i == NB - 1)
    def _():
        d2_ref[0] = d2_ref[0] + n2


def _tc_pair(x1, x2t, nb_batches):
    return pl.pallas_call(
        _tc_body,
        grid=(nb_batches, NB),
        in_specs=[
            pl.BlockSpec((1, TILE_N, 3), lambda b, i: (b, i, 0)),
            pl.BlockSpec((1, 3, N), lambda b, i: (b, 0, 0)),
        ],
        out_specs=[
            pl.BlockSpec((1, 1, 1, TILE_N), lambda b, i: (b, i, 0, 0)),
            pl.BlockSpec((1, 1, N), lambda b, i: (b, 0, 0)),
        ],
        out_shape=[
            jax.ShapeDtypeStruct((nb_batches, NB, 1, TILE_N), jnp.float32),
            jax.ShapeDtypeStruct((nb_batches, 1, N), jnp.float32),
        ],
        compiler_params=pltpu.CompilerParams(
            dimension_semantics=("parallel", "arbitrary"),
        ),
    )(x1, x2t)


def kernel(xyz1, xyz2):
    outs1 = []
    outs2 = []
    if KSC:
        pts = jnp.stack([
            jnp.moveaxis(xyz1[:KSC], -1, 0),
            jnp.moveaxis(xyz2[:KSC], -1, 0),
        ])
        sc_out = _make_sc_kernel()(pts)
        outs1.append(sc_out[0])
        outs2.append(sc_out[1])
    if KSC < B:
        ntc = B - KSC
        x1 = xyz1[KSC:]
        x2 = xyz2[KSC:]
        x2t = jnp.moveaxis(x2, -1, 1)
        d1, d2 = _tc_pair(x1, x2t, ntc)
        outs1.append(d1.reshape(ntc, N))
        outs2.append(d2.reshape(ntc, N))
    dist1 = outs1[0] if len(outs1) == 1 else jnp.concatenate(outs1, axis=0)
    dist2 = outs2[0] if len(outs2) == 1 else jnp.concatenate(outs2, axis=0)
    return dist1, dist2


# speedup vs baseline: 6.0588x; 1.1738x over previous
import functools

import jax
import jax.numpy as jnp
from jax import lax
from jax.experimental import pallas as pl
from jax.experimental.pallas import tpu as pltpu
from jax.experimental.pallas import tpu_sc as plsc

B = 8
N = 2048
L = 16
NKV = N // L
QBLK = 4
UNROLL = 4

KSC = 0

TILE_N = 256
NB = N // TILE_N


_CHUNKS = 16 // KSC if KSC else 1
_QLEN = N // _CHUNKS if KSC else N


def _round_bf16(v):
    u = plsc.bitcast(v, jnp.uint32)
    lsb = (u >> jnp.uint32(16)) & jnp.uint32(1)
    r = (u + jnp.uint32(0x7FFF) + lsb) & jnp.uint32(0xFFFF0000)
    return plsc.bitcast(r, jnp.float32)


def _sc_tile_body(pts_hbm, out_hbm, qx, qy, qz, q2, kx, ky, kz, k2, res):
    c = lax.axis_index("c")
    s = lax.axis_index("s")
    w = s * 2 + c
    d = w // 16
    r = w % 16
    b = r // _CHUNKS
    h = r % _CHUNKS
    q0 = h * _QLEN
    dk = 1 - d

    pltpu.sync_copy(pts_hbm.at[d, 0, b, pl.ds(q0, _QLEN)], qx)
    pltpu.sync_copy(pts_hbm.at[d, 1, b, pl.ds(q0, _QLEN)], qy)
    pltpu.sync_copy(pts_hbm.at[d, 2, b, pl.ds(q0, _QLEN)], qz)
    pltpu.sync_copy(pts_hbm.at[dk, 0, b], kx)
    pltpu.sync_copy(pts_hbm.at[dk, 1, b], ky)
    pltpu.sync_copy(pts_hbm.at[dk, 2, b], kz)

    def k2_body(j, carry):
        off = j * L
        xs = kx[pl.ds(off, L)]
        ys = ky[pl.ds(off, L)]
        zs = kz[pl.ds(off, L)]
        k2[pl.ds(off, L)] = xs * xs + ys * ys + zs * zs
        kx[pl.ds(off, L)] = _round_bf16(xs)
        ky[pl.ds(off, L)] = _round_bf16(ys)
        kz[pl.ds(off, L)] = _round_bf16(zs)
        return carry

    lax.fori_loop(0, NKV, k2_body, 0)

    def q2_body(j, carry):
        off = j * L
        xs = qx[pl.ds(off, L)]
        ys = qy[pl.ds(off, L)]
        zs = qz[pl.ds(off, L)]
        q2[pl.ds(off, L)] = xs * xs + ys * ys + zs * zs
        qx[pl.ds(off, L)] = _round_bf16(xs)
        qy[pl.ds(off, L)] = _round_bf16(ys)
        qz[pl.ds(off, L)] = _round_bf16(zs)
        return carry

    lax.fori_loop(0, _QLEN // L, q2_body, 0)

    inf = jnp.full((L,), jnp.inf, jnp.float32)

    def qblk_body(i, carry):
        qbase = i * QBLK
        av = []
        bv = []
        cv = []
        q2v = []
        for u in range(QBLK):
            idx = jnp.full((L,), qbase + u, jnp.int32)
            av.append(-2.0 * plsc.load_gather(qx, [idx]))
            bv.append(-2.0 * plsc.load_gather(qy, [idx]))
            cv.append(-2.0 * plsc.load_gather(qz, [idx]))
            q2v.append(plsc.load_gather(q2, [idx]))

        def k_body(jj, accs):
            accs = list(accs)
            for t in range(UNROLL):
                off = (jj * UNROLL + t) * L
                xk = kx[pl.ds(off, L)]
                yk = ky[pl.ds(off, L)]
                zk = kz[pl.ds(off, L)]
                kk = k2[pl.ds(off, L)]
                for u in range(QBLK):
                    dv = kk + av[u] * xk + bv[u] * yk + cv[u] * zk
                    accs[u] = jnp.minimum(accs[u], dv)
            return tuple(accs)

        accs = lax.fori_loop(0, NKV // UNROLL, k_body,
                             (inf,) * QBLK)
        lane0 = lax.iota(jnp.int32, L) == 0
        for u in range(QBLK):
            m = jnp.min(accs[u] + q2v[u])
            idx = jnp.full((L,), qbase + u, jnp.int32)
            plsc.store_scatter(res, [idx], jnp.full((L,), m), mask=lane0)
        return carry

    lax.fori_loop(0, _QLEN // QBLK, qblk_body, 0)

    pltpu.sync_copy(res, out_hbm.at[d, b, pl.ds(q0, _QLEN)])


def _make_sc_kernel():
    mesh = plsc.VectorSubcoreMesh(core_axis_name="c", subcore_axis_name="s")
    return pl.kernel(
        _sc_tile_body,
        out_type=jax.ShapeDtypeStruct((2, KSC, N), jnp.float32),
        mesh=mesh,
        scratch_types=[
            pltpu.VMEM((_QLEN,), jnp.float32),
            pltpu.VMEM((_QLEN,), jnp.float32),
            pltpu.VMEM((_QLEN,), jnp.float32),
            pltpu.VMEM((_QLEN,), jnp.float32),
            pltpu.VMEM((N,), jnp.float32),
            pltpu.VMEM((N,), jnp.float32),
            pltpu.VMEM((N,), jnp.float32),
            pltpu.VMEM((N,), jnp.float32),
            pltpu.VMEM((_QLEN,), jnp.float32),
        ],
        compiler_params=pltpu.CompilerParams(needs_layout_passes=False),
    )


def _tc_body(x1_ref, p2t_ref, d1_ref, d2_ref):
    i = pl.program_id(1)
    xb = x1_ref[0]
    yt = p2t_ref[0]
    n1 = jnp.sum(xb * xb, axis=1, keepdims=True)
    n2 = jnp.sum(yt * yt, axis=0, keepdims=True)
    a = (-2.0 * xb).astype(jnp.bfloat16)
    yr = yt.astype(jnp.bfloat16)
    t = lax.dot_general(a, yr, (((1,), (0,)), ((), ())),
                        preferred_element_type=jnp.float32)
    d1_ref[0, 0, 0, :] = jnp.min(t + n2, axis=1) + n1[:, 0]
    v = jnp.min(t + n1, axis=0, keepdims=True)

    @pl.when(i == 0)
    def _():
        d2_ref[0] = v

    @pl.when(i > 0)
    def _():
        d2_ref[0] = jnp.minimum(d2_ref[0], v)

    @pl.when(i == NB - 1)
    def _():
        d2_ref[0] = d2_ref[0] + n2


def _tc_pair(x1, x2t, nb_batches):
    return pl.pallas_call(
        _tc_body,
        grid=(nb_batches, NB),
        in_specs=[
            pl.BlockSpec((1, TILE_N, 3), lambda b, i: (b, i, 0)),
            pl.BlockSpec((1, 3, N), lambda b, i: (b, 0, 0)),
        ],
        out_specs=[
            pl.BlockSpec((1, 1, 1, TILE_N), lambda b, i: (b, i, 0, 0)),
            pl.BlockSpec((1, 1, N), lambda b, i: (b, 0, 0)),
        ],
        out_shape=[
            jax.ShapeDtypeStruct((nb_batches, NB, 1, TILE_N), jnp.float32),
            jax.ShapeDtypeStruct((nb_batches, 1, N), jnp.float32),
        ],
        compiler_params=pltpu.CompilerParams(
            dimension_semantics=("parallel", "arbitrary"),
        ),
    )(x1, x2t)


def kernel(xyz1, xyz2):
    outs1 = []
    outs2 = []
    if KSC:
        pts = jnp.stack([
            jnp.moveaxis(xyz1[:KSC], -1, 0),
            jnp.moveaxis(xyz2[:KSC], -1, 0),
        ])
        sc_out = _make_sc_kernel()(pts)
        outs1.append(sc_out[0])
        outs2.append(sc_out[1])
    if KSC < B:
        ntc = B - KSC
        x1 = xyz1[KSC:]
        x2 = xyz2[KSC:]
        x2t = jnp.moveaxis(x2, -1, 1)
        d1, d2 = _tc_pair(x1, x2t, ntc)
        outs1.append(d1.reshape(ntc, N))
        outs2.append(d2.reshape(ntc, N))
    dist1 = outs1[0] if len(outs1) == 1 else jnp.concatenate(outs1, axis=0)
    dist2 = outs2[0] if len(outs2) == 1 else jnp.concatenate(outs2, axis=0)
    return dist1, dist2


# speedup vs baseline: 6.7603x; 1.1158x over previous
import functools

import jax
import jax.numpy as jnp
from jax import lax
from jax.experimental import pallas as pl
from jax.experimental.pallas import tpu as pltpu
from jax.experimental.pallas import tpu_sc as plsc

B = 8
N = 2048
L = 16
NKV = N // L
QBLK = 4
UNROLL = 4

KSC = 0

TILE_N = 2048
NB = N // TILE_N


_CHUNKS = 16 // KSC if KSC else 1
_QLEN = N // _CHUNKS if KSC else N


def _round_bf16(v):
    u = plsc.bitcast(v, jnp.uint32)
    lsb = (u >> jnp.uint32(16)) & jnp.uint32(1)
    r = (u + jnp.uint32(0x7FFF) + lsb) & jnp.uint32(0xFFFF0000)
    return plsc.bitcast(r, jnp.float32)


def _sc_tile_body(pts_hbm, out_hbm, qx, qy, qz, q2, kx, ky, kz, k2, res):
    c = lax.axis_index("c")
    s = lax.axis_index("s")
    w = s * 2 + c
    d = w // 16
    r = w % 16
    b = r // _CHUNKS
    h = r % _CHUNKS
    q0 = h * _QLEN
    dk = 1 - d

    pltpu.sync_copy(pts_hbm.at[d, 0, b, pl.ds(q0, _QLEN)], qx)
    pltpu.sync_copy(pts_hbm.at[d, 1, b, pl.ds(q0, _QLEN)], qy)
    pltpu.sync_copy(pts_hbm.at[d, 2, b, pl.ds(q0, _QLEN)], qz)
    pltpu.sync_copy(pts_hbm.at[dk, 0, b], kx)
    pltpu.sync_copy(pts_hbm.at[dk, 1, b], ky)
    pltpu.sync_copy(pts_hbm.at[dk, 2, b], kz)

    def k2_body(j, carry):
        off = j * L
        xs = kx[pl.ds(off, L)]
        ys = ky[pl.ds(off, L)]
        zs = kz[pl.ds(off, L)]
        k2[pl.ds(off, L)] = xs * xs + ys * ys + zs * zs
        kx[pl.ds(off, L)] = _round_bf16(xs)
        ky[pl.ds(off, L)] = _round_bf16(ys)
        kz[pl.ds(off, L)] = _round_bf16(zs)
        return carry

    lax.fori_loop(0, NKV, k2_body, 0)

    def q2_body(j, carry):
        off = j * L
        xs = qx[pl.ds(off, L)]
        ys = qy[pl.ds(off, L)]
        zs = qz[pl.ds(off, L)]
        q2[pl.ds(off, L)] = xs * xs + ys * ys + zs * zs
        qx[pl.ds(off, L)] = _round_bf16(xs)
        qy[pl.ds(off, L)] = _round_bf16(ys)
        qz[pl.ds(off, L)] = _round_bf16(zs)
        return carry

    lax.fori_loop(0, _QLEN // L, q2_body, 0)

    inf = jnp.full((L,), jnp.inf, jnp.float32)

    def qblk_body(i, carry):
        qbase = i * QBLK
        av = []
        bv = []
        cv = []
        q2v = []
        for u in range(QBLK):
            idx = jnp.full((L,), qbase + u, jnp.int32)
            av.append(-2.0 * plsc.load_gather(qx, [idx]))
            bv.append(-2.0 * plsc.load_gather(qy, [idx]))
            cv.append(-2.0 * plsc.load_gather(qz, [idx]))
            q2v.append(plsc.load_gather(q2, [idx]))

        def k_body(jj, accs):
            accs = list(accs)
            for t in range(UNROLL):
                off = (jj * UNROLL + t) * L
                xk = kx[pl.ds(off, L)]
                yk = ky[pl.ds(off, L)]
                zk = kz[pl.ds(off, L)]
                kk = k2[pl.ds(off, L)]
                for u in range(QBLK):
                    dv = kk + av[u] * xk + bv[u] * yk + cv[u] * zk
                    accs[u] = jnp.minimum(accs[u], dv)
            return tuple(accs)

        accs = lax.fori_loop(0, NKV // UNROLL, k_body,
                             (inf,) * QBLK)
        lane0 = lax.iota(jnp.int32, L) == 0
        for u in range(QBLK):
            m = jnp.min(accs[u] + q2v[u])
            idx = jnp.full((L,), qbase + u, jnp.int32)
            plsc.store_scatter(res, [idx], jnp.full((L,), m), mask=lane0)
        return carry

    lax.fori_loop(0, _QLEN // QBLK, qblk_body, 0)

    pltpu.sync_copy(res, out_hbm.at[d, b, pl.ds(q0, _QLEN)])


def _make_sc_kernel():
    mesh = plsc.VectorSubcoreMesh(core_axis_name="c", subcore_axis_name="s")
    return pl.kernel(
        _sc_tile_body,
        out_type=jax.ShapeDtypeStruct((2, KSC, N), jnp.float32),
        mesh=mesh,
        scratch_types=[
            pltpu.VMEM((_QLEN,), jnp.float32),
            pltpu.VMEM((_QLEN,), jnp.float32),
            pltpu.VMEM((_QLEN,), jnp.float32),
            pltpu.VMEM((_QLEN,), jnp.float32),
            pltpu.VMEM((N,), jnp.float32),
            pltpu.VMEM((N,), jnp.float32),
            pltpu.VMEM((N,), jnp.float32),
            pltpu.VMEM((N,), jnp.float32),
            pltpu.VMEM((_QLEN,), jnp.float32),
        ],
        compiler_params=pltpu.CompilerParams(needs_layout_passes=False),
    )


def _tc_body(x1_ref, p2t_ref, d1_ref, d2_ref):
    i = pl.program_id(1)
    xb = x1_ref[0]
    yt = p2t_ref[0]
    n1 = jnp.sum(xb * xb, axis=1, keepdims=True)
    n2 = jnp.sum(yt * yt, axis=0, keepdims=True)
    a = (-2.0 * xb).astype(jnp.bfloat16)
    yr = yt.astype(jnp.bfloat16)
    t = lax.dot_general(a, yr, (((1,), (0,)), ((), ())),
                        preferred_element_type=jnp.float32)
    d1_ref[0, 0, 0, :] = jnp.min(t + n2, axis=1) + n1[:, 0]
    v = jnp.min(t + n1, axis=0, keepdims=True)

    @pl.when(i == 0)
    def _():
        d2_ref[0] = v

    @pl.when(i > 0)
    def _():
        d2_ref[0] = jnp.minimum(d2_ref[0], v)

    @pl.when(i == NB - 1)
    def _():
        d2_ref[0] = d2_ref[0] + n2


def _tc_pair(x1, x2t, nb_batches):
    return pl.pallas_call(
        _tc_body,
        grid=(nb_batches, NB),
        in_specs=[
            pl.BlockSpec((1, TILE_N, 3), lambda b, i: (b, i, 0)),
            pl.BlockSpec((1, 3, N), lambda b, i: (b, 0, 0)),
        ],
        out_specs=[
            pl.BlockSpec((1, 1, 1, TILE_N), lambda b, i: (b, i, 0, 0)),
            pl.BlockSpec((1, 1, N), lambda b, i: (b, 0, 0)),
        ],
        out_shape=[
            jax.ShapeDtypeStruct((nb_batches, NB, 1, TILE_N), jnp.float32),
            jax.ShapeDtypeStruct((nb_batches, 1, N), jnp.float32),
        ],
        compiler_params=pltpu.CompilerParams(
            dimension_semantics=("parallel", "arbitrary"),
        ),
    )(x1, x2t)


def kernel(xyz1, xyz2):
    outs1 = []
    outs2 = []
    if KSC:
        pts = jnp.stack([
            jnp.moveaxis(xyz1[:KSC], -1, 0),
            jnp.moveaxis(xyz2[:KSC], -1, 0),
        ])
        sc_out = _make_sc_kernel()(pts)
        outs1.append(sc_out[0])
        outs2.append(sc_out[1])
    if KSC < B:
        ntc = B - KSC
        x1 = xyz1[KSC:]
        x2 = xyz2[KSC:]
        x2t = jnp.moveaxis(x2, -1, 1)
        d1, d2 = _tc_pair(x1, x2t, ntc)
        outs1.append(d1.reshape(ntc, N))
        outs2.append(d2.reshape(ntc, N))
    dist1 = outs1[0] if len(outs1) == 1 else jnp.concatenate(outs1, axis=0)
    dist2 = outs2[0] if len(outs2) == 1 else jnp.concatenate(outs2, axis=0)
    return dist1, dist2


# speedup vs baseline: 7.0934x; 1.0493x over previous
import functools

import jax
import jax.numpy as jnp
from jax import lax
from jax.experimental import pallas as pl
from jax.experimental.pallas import tpu as pltpu
from jax.experimental.pallas import tpu_sc as plsc

B = 8
N = 2048
L = 16
NKV = N // L
QBLK = 4
UNROLL = 4

KSC = 0

TILE_N = 2048
NB = N // TILE_N


_CHUNKS = 16 // KSC if KSC else 1
_QLEN = N // _CHUNKS if KSC else N


def _round_bf16(v):
    u = plsc.bitcast(v, jnp.uint32)
    lsb = (u >> jnp.uint32(16)) & jnp.uint32(1)
    r = (u + jnp.uint32(0x7FFF) + lsb) & jnp.uint32(0xFFFF0000)
    return plsc.bitcast(r, jnp.float32)


def _sc_tile_body(pts_hbm, out_hbm, qx, qy, qz, q2, kx, ky, kz, k2, res):
    c = lax.axis_index("c")
    s = lax.axis_index("s")
    w = s * 2 + c
    d = w // 16
    r = w % 16
    b = r // _CHUNKS
    h = r % _CHUNKS
    q0 = h * _QLEN
    dk = 1 - d

    pltpu.sync_copy(pts_hbm.at[d, 0, b, pl.ds(q0, _QLEN)], qx)
    pltpu.sync_copy(pts_hbm.at[d, 1, b, pl.ds(q0, _QLEN)], qy)
    pltpu.sync_copy(pts_hbm.at[d, 2, b, pl.ds(q0, _QLEN)], qz)
    pltpu.sync_copy(pts_hbm.at[dk, 0, b], kx)
    pltpu.sync_copy(pts_hbm.at[dk, 1, b], ky)
    pltpu.sync_copy(pts_hbm.at[dk, 2, b], kz)

    def k2_body(j, carry):
        off = j * L
        xs = kx[pl.ds(off, L)]
        ys = ky[pl.ds(off, L)]
        zs = kz[pl.ds(off, L)]
        k2[pl.ds(off, L)] = xs * xs + ys * ys + zs * zs
        kx[pl.ds(off, L)] = _round_bf16(xs)
        ky[pl.ds(off, L)] = _round_bf16(ys)
        kz[pl.ds(off, L)] = _round_bf16(zs)
        return carry

    lax.fori_loop(0, NKV, k2_body, 0)

    def q2_body(j, carry):
        off = j * L
        xs = qx[pl.ds(off, L)]
        ys = qy[pl.ds(off, L)]
        zs = qz[pl.ds(off, L)]
        q2[pl.ds(off, L)] = xs * xs + ys * ys + zs * zs
        qx[pl.ds(off, L)] = _round_bf16(xs)
        qy[pl.ds(off, L)] = _round_bf16(ys)
        qz[pl.ds(off, L)] = _round_bf16(zs)
        return carry

    lax.fori_loop(0, _QLEN // L, q2_body, 0)

    inf = jnp.full((L,), jnp.inf, jnp.float32)

    def qblk_body(i, carry):
        qbase = i * QBLK
        av = []
        bv = []
        cv = []
        q2v = []
        for u in range(QBLK):
            idx = jnp.full((L,), qbase + u, jnp.int32)
            av.append(-2.0 * plsc.load_gather(qx, [idx]))
            bv.append(-2.0 * plsc.load_gather(qy, [idx]))
            cv.append(-2.0 * plsc.load_gather(qz, [idx]))
            q2v.append(plsc.load_gather(q2, [idx]))

        def k_body(jj, accs):
            accs = list(accs)
            for t in range(UNROLL):
                off = (jj * UNROLL + t) * L
                xk = kx[pl.ds(off, L)]
                yk = ky[pl.ds(off, L)]
                zk = kz[pl.ds(off, L)]
                kk = k2[pl.ds(off, L)]
                for u in range(QBLK):
                    dv = kk + av[u] * xk + bv[u] * yk + cv[u] * zk
                    accs[u] = jnp.minimum(accs[u], dv)
            return tuple(accs)

        accs = lax.fori_loop(0, NKV // UNROLL, k_body,
                             (inf,) * QBLK)
        lane0 = lax.iota(jnp.int32, L) == 0
        for u in range(QBLK):
            m = jnp.min(accs[u] + q2v[u])
            idx = jnp.full((L,), qbase + u, jnp.int32)
            plsc.store_scatter(res, [idx], jnp.full((L,), m), mask=lane0)
        return carry

    lax.fori_loop(0, _QLEN // QBLK, qblk_body, 0)

    pltpu.sync_copy(res, out_hbm.at[d, b, pl.ds(q0, _QLEN)])


def _make_sc_kernel():
    mesh = plsc.VectorSubcoreMesh(core_axis_name="c", subcore_axis_name="s")
    return pl.kernel(
        _sc_tile_body,
        out_type=jax.ShapeDtypeStruct((2, KSC, N), jnp.float32),
        mesh=mesh,
        scratch_types=[
            pltpu.VMEM((_QLEN,), jnp.float32),
            pltpu.VMEM((_QLEN,), jnp.float32),
            pltpu.VMEM((_QLEN,), jnp.float32),
            pltpu.VMEM((_QLEN,), jnp.float32),
            pltpu.VMEM((N,), jnp.float32),
            pltpu.VMEM((N,), jnp.float32),
            pltpu.VMEM((N,), jnp.float32),
            pltpu.VMEM((N,), jnp.float32),
            pltpu.VMEM((_QLEN,), jnp.float32),
        ],
        compiler_params=pltpu.CompilerParams(needs_layout_passes=False),
    )


def _tc_body(x1_ref, p2t_ref, d1_ref, d2_ref):
    xb = x1_ref[0]
    yt = p2t_ref[0]
    n1 = jnp.sum(xb * xb, axis=1, keepdims=True)
    n2 = jnp.sum(yt * yt, axis=0, keepdims=True)
    hi = n2.astype(jnp.bfloat16)
    r1 = n2 - hi.astype(jnp.float32)
    mid = r1.astype(jnp.bfloat16)
    lo = (r1 - mid.astype(jnp.float32)).astype(jnp.bfloat16)
    a = (-2.0 * xb).astype(jnp.bfloat16)
    ones = jnp.ones((TILE_N, 3), jnp.bfloat16)
    lhs = jnp.concatenate([a, ones], axis=1)
    rhs = jnp.concatenate(
        [yt.astype(jnp.bfloat16), hi, mid, lo], axis=0)
    t = lax.dot_general(lhs, rhs, (((1,), (0,)), ((), ())),
                        preferred_element_type=jnp.float32)
    d1_ref[0, 0, 0, :] = jnp.min(t, axis=1) + n1[:, 0]
    d2_ref[0] = jnp.min(t + n1, axis=0, keepdims=True)


def _tc_pair(x1, x2t, nb_batches):
    return pl.pallas_call(
        _tc_body,
        grid=(nb_batches, NB),
        in_specs=[
            pl.BlockSpec((1, TILE_N, 3), lambda b, i: (b, i, 0)),
            pl.BlockSpec((1, 3, N), lambda b, i: (b, 0, 0)),
        ],
        out_specs=[
            pl.BlockSpec((1, 1, 1, TILE_N), lambda b, i: (b, i, 0, 0)),
            pl.BlockSpec((1, 1, N), lambda b, i: (b, 0, 0)),
        ],
        out_shape=[
            jax.ShapeDtypeStruct((nb_batches, NB, 1, TILE_N), jnp.float32),
            jax.ShapeDtypeStruct((nb_batches, 1, N), jnp.float32),
        ],
        compiler_params=pltpu.CompilerParams(
            dimension_semantics=("parallel", "arbitrary"),
        ),
    )(x1, x2t)


def kernel(xyz1, xyz2):
    outs1 = []
    outs2 = []
    if KSC:
        pts = jnp.stack([
            jnp.moveaxis(xyz1[:KSC], -1, 0),
            jnp.moveaxis(xyz2[:KSC], -1, 0),
        ])
        sc_out = _make_sc_kernel()(pts)
        outs1.append(sc_out[0])
        outs2.append(sc_out[1])
    if KSC < B:
        ntc = B - KSC
        x1 = xyz1[KSC:]
        x2 = xyz2[KSC:]
        x2t = jnp.moveaxis(x2, -1, 1)
        d1, d2 = _tc_pair(x1, x2t, ntc)
        outs1.append(d1.reshape(ntc, N))
        outs2.append(d2.reshape(ntc, N))
    dist1 = outs1[0] if len(outs1) == 1 else jnp.concatenate(outs1, axis=0)
    dist2 = outs2[0] if len(outs2) == 1 else jnp.concatenate(outs2, axis=0)
    return dist1, dist2
